# baseline (speedup 1.0000x reference)
import functools

import jax
import jax.numpy as jnp
from jax import lax
from jax.experimental import pallas as pl
from jax.experimental.pallas import tpu as pltpu
from jax.experimental.pallas import tpu_sc as plsc

MAXP = 32
NIDX = 2 * MAXP + 1
ZD = 128
LS = 512
NC, NS = 2, 16
NW = NC * NS
RPW = LS // NW
WIN = LS + RPW

_mesh = plsc.VectorSubcoreMesh(core_axis_name="c", subcore_axis_name="s")


@functools.partial(
    pl.kernel,
    mesh=_mesh,
    out_type=jax.ShapeDtypeStruct((LS, LS, ZD), jnp.float32),
    scratch_types=[
        pltpu.VMEM((NIDX, ZD), jnp.float32),
        pltpu.VMEM((WIN, ZD), jnp.float32),
        pltpu.SemaphoreType.DMA,
    ],
)
def _rel_pos_bias(table_hbm, out_hbm, table_v, win_v, sem):
    wid = lax.axis_index("s") * NC + lax.axis_index("c")
    row0 = wid * RPW
    m0 = (LS - 1) - (row0 + (RPW - 1))

    pltpu.sync_copy(table_hbm, table_v)

    QTR = LS // 4

    copies = []
    for q in range(4):
        lo, hi = q * QTR, q * QTR + QTR + RPW if q < 3 else WIN

        @plsc.parallel_loop(q * QTR + (RPW if q else 0), hi, 1, unroll=8)
        def _build(r):
            c = jnp.clip((LS - 1) - (m0 + r), -MAXP, MAXP) + MAXP
            for k in range(ZD // 16):
                win_v[r, pl.ds(k * 16, 16)] = table_v[c, pl.ds(k * 16, 16)]

        for t in range(RPW):
            cp = pltpu.make_async_copy(
                win_v.at[pl.ds((RPW - 1) - t + q * QTR, QTR)],
                out_hbm.at[row0 + t, pl.ds(q * QTR, QTR)],
                sem,
            )
            cp.start()
            copies.append(cp)
    for cp in copies:
        cp.wait()


def kernel(L, embed_table):
    return _rel_pos_bias(embed_table)


# speedup vs baseline: 1.0232x; 1.0232x over previous
import functools

import jax
import jax.numpy as jnp
from jax import lax
from jax.experimental import pallas as pl
from jax.experimental.pallas import tpu as pltpu
from jax.experimental.pallas import tpu_sc as plsc

MAXP = 32
NIDX = 2 * MAXP + 1
ZD = 128
LS = 512
NC, NS = 2, 16
NW = NC * NS
RPW = LS // NW
WIN = LS + RPW

_mesh = plsc.VectorSubcoreMesh(core_axis_name="c", subcore_axis_name="s")


@functools.partial(
    pl.kernel,
    mesh=_mesh,
    out_type=jax.ShapeDtypeStruct((LS, LS, ZD), jnp.float32),
    scratch_types=[
        pltpu.VMEM((NIDX, ZD), jnp.float32),
        pltpu.VMEM((WIN, ZD), jnp.float32),
        pltpu.SemaphoreType.DMA,
    ],
)
def _rel_pos_bias(table_hbm, out_hbm, table_v, win_v, sem):
    wid = lax.axis_index("s") * NC + lax.axis_index("c")
    row0 = wid * RPW
    m0 = (LS - 1) - (row0 + (RPW - 1))

    pltpu.sync_copy(table_hbm, table_v)

    HALF = LS // 2
    CUT = HALF + RPW

    @plsc.parallel_loop(0, CUT, 1, unroll=8)
    def _build_lo(r):
        c = jnp.clip((LS - 1) - (m0 + r), -MAXP, MAXP) + MAXP
        for k in range(ZD // 16):
            win_v[r, pl.ds(k * 16, 16)] = table_v[c, pl.ds(k * 16, 16)]

    copies = []
    for t in range(RPW):
        cp = pltpu.make_async_copy(
            win_v.at[pl.ds((RPW - 1) - t, HALF)],
            out_hbm.at[row0 + t, pl.ds(0, HALF)],
            sem,
        )
        cp.start()
        copies.append(cp)

    @plsc.parallel_loop(CUT, WIN, 1, unroll=8)
    def _build_hi(r):
        c = jnp.clip((LS - 1) - (m0 + r), -MAXP, MAXP) + MAXP
        for k in range(ZD // 16):
            win_v[r, pl.ds(k * 16, 16)] = table_v[c, pl.ds(k * 16, 16)]

    for t in range(RPW):
        cp = pltpu.make_async_copy(
            win_v.at[pl.ds((RPW - 1) - t + HALF, HALF)],
            out_hbm.at[row0 + t, pl.ds(HALF, HALF)],
            sem,
        )
        cp.start()
        copies.append(cp)
    for cp in copies:
        cp.wait()


def kernel(L, embed_table):
    return _rel_pos_bias(embed_table)
